# baseline (speedup 1.0000x reference)
import functools

import jax
import jax.numpy as jnp
from jax import lax
from jax.experimental import pallas as pl
from jax.experimental.pallas import tpu as pltpu

B, L, D = 16, 2048, 1024
CH = 256
NBUF = 2


def _body(len_ref, in_hbm, out_ref, buf, sem):
    i = pl.program_id(0)
    n = len_ref[i]
    nchunks = lax.div(n + (CH - 1), CH)

    def chunk_copy(k, slot):
        start = pl.multiple_of(k * CH, CH)
        return pltpu.make_async_copy(
            in_hbm.at[i, pl.ds(start, CH), :],
            buf.at[slot],
            sem.at[slot],
        )

    chunk_copy(0, 0).start()

    def step(k, acc):
        slot = lax.rem(k, NBUF)

        @pl.when(k + 1 < nchunks)
        def _():
            chunk_copy(k + 1, lax.rem(k + 1, NBUF)).start()

        chunk_copy(k, slot).wait()
        rows_left = n - k * CH
        row_id = lax.broadcasted_iota(jnp.int32, (CH, 1), 0)
        data = jnp.where(row_id < rows_left, buf[slot], 0.0)
        return acc + jnp.sum(data, axis=0)

    acc = lax.fori_loop(0, nchunks, step, jnp.zeros((D,), jnp.float32))
    out_ref[i, :] = acc / n.astype(jnp.float32)


def kernel(input, length):
    length = length.astype(jnp.int32)
    grid_spec = pltpu.PrefetchScalarGridSpec(
        num_scalar_prefetch=1,
        grid=(B,),
        in_specs=[pl.BlockSpec(memory_space=pl.ANY)],
        out_specs=pl.BlockSpec((B, D), lambda i, s: (0, 0)),
        scratch_shapes=[
            pltpu.VMEM((NBUF, CH, D), jnp.float32),
            pltpu.SemaphoreType.DMA((NBUF,)),
        ],
    )
    return pl.pallas_call(
        _body,
        grid_spec=grid_spec,
        out_shape=jax.ShapeDtypeStruct((B, D), jnp.float32),
    )(length, input)
